# baseline (speedup 1.0000x reference)
import functools

import jax
import jax.numpy as jnp
from jax.experimental import pallas as pl
from jax.experimental.pallas import tpu as pltpu

NSEG = 128


def _fused_kernel(labels_ref, x0_ref, x1_ref, ww_ref, wb_ref, vw_ref,
                  out_ref):
    s_len = labels_ref.shape[-1]
    half = s_len // 2
    ids = labels_ref[0, 0, :].reshape(1, s_len)
    seg = jax.lax.broadcasted_iota(jnp.int32, (NSEG, s_len), 0)
    onehot = seg == ids
    p = onehot.astype(jnp.bfloat16)
    sums = jnp.dot(p[:, :half], x0_ref[0, 0].astype(jnp.bfloat16),
                   preferred_element_type=jnp.float32)
    sums += jnp.dot(p[:, half:], x1_ref[0, 0].astype(jnp.bfloat16),
                    preferred_element_type=jnp.float32)
    cnt = jnp.sum(onehot.astype(jnp.float32), axis=1, keepdims=True)
    rec = 1.0 / jnp.maximum(cnt, 1e-12)
    f = sums * rec
    att = jax.lax.dot_general(
        f.astype(jnp.bfloat16), ww_ref[...].astype(jnp.bfloat16),
        (((1,), (1,)), ((), ())),
        preferred_element_type=jnp.float32)
    att = jnp.tanh(att + wb_ref[...])
    score = jax.lax.dot_general(
        att, vw_ref[...], (((1,), (1,)), ((), ())),
        preferred_element_type=jnp.float32)
    n = labels_ref[0, 0, s_len - 1] + 1
    segc = jax.lax.broadcasted_iota(jnp.int32, (NSEG, 1), 0)
    valid = jnp.logical_and(segc >= 1, segc < n)
    score = jnp.where(valid, score, jnp.float32(-10000.0))
    m = jnp.max(score, axis=0, keepdims=True)
    e = jnp.exp(score - m)
    w = e / jnp.sum(e, axis=0, keepdims=True)
    w = w * valid.astype(jnp.float32)
    out_ref[0] = jax.lax.dot_general(
        w, f, (((0,), (0,)), ((), ())),
        preferred_element_type=jnp.float32)


@functools.partial(jax.jit, static_argnames=("interpret",))
def _run(last_hidden_state, labels3, W_w, W_b2, V_w, interpret=False):
    B, S, D = last_hidden_state.shape
    H = W_w.shape[0]
    half = S // 2
    x4 = last_hidden_state.reshape(B, 2, half, D)

    ctx = pl.pallas_call(
        _fused_kernel,
        grid=(B,),
        in_specs=[
            pl.BlockSpec((1, 1, S), lambda i: (i, 0, 0)),
            pl.BlockSpec((1, 1, half, D), lambda i: (i, 0, 0, 0)),
            pl.BlockSpec((1, 1, half, D), lambda i: (i, 1, 0, 0)),
            pl.BlockSpec((H, D), lambda i: (0, 0)),
            pl.BlockSpec((1, H), lambda i: (0, 0)),
            pl.BlockSpec((1, H), lambda i: (0, 0)),
        ],
        out_specs=pl.BlockSpec((1, 1, D), lambda i: (i, 0, 0)),
        out_shape=jax.ShapeDtypeStruct((B, 1, D), jnp.float32),
        compiler_params=pltpu.CompilerParams(
            dimension_semantics=("arbitrary",)),
        interpret=interpret,
    )(labels3, x4, x4, W_w, W_b2, V_w)
    return ctx.reshape(B, D)


def kernel(last_hidden_state, labeled_input_ids, W_w, W_b, V_w, V_b):
    B, S, D = last_hidden_state.shape
    H = W_w.shape[0]
    labels3 = labeled_input_ids.astype(jnp.int32).reshape(B, 1, S)
    return _run(last_hidden_state, labels3, W_w, W_b.reshape(1, H), V_w)


# speedup vs baseline: 1.0413x; 1.0413x over previous
import functools

import jax
import jax.numpy as jnp
from jax.experimental import pallas as pl
from jax.experimental.pallas import tpu as pltpu

NSEG = 128


def _fused_kernel(labels_ref, x_ref, ww_ref, wb_ref, vw_ref, out_ref):
    s_len = labels_ref.shape[-1]
    ids = labels_ref[0, 0, :].reshape(1, s_len)
    seg = jax.lax.broadcasted_iota(jnp.int32, (NSEG, s_len), 0)
    onehot = seg == ids
    p = onehot.astype(jnp.bfloat16)
    sums = jnp.dot(p, x_ref[0].astype(jnp.bfloat16),
                   preferred_element_type=jnp.float32)
    cnt = jnp.sum(onehot.astype(jnp.float32), axis=1, keepdims=True)
    rec = 1.0 / jnp.maximum(cnt, 1e-12)
    f = sums * rec
    att = jax.lax.dot_general(
        f.astype(jnp.bfloat16), ww_ref[...].astype(jnp.bfloat16),
        (((1,), (1,)), ((), ())),
        preferred_element_type=jnp.float32)
    att = jnp.tanh(att + wb_ref[...])
    score = jax.lax.dot_general(
        att, vw_ref[...], (((1,), (1,)), ((), ())),
        preferred_element_type=jnp.float32)
    n = labels_ref[0, 0, s_len - 1] + 1
    segc = jax.lax.broadcasted_iota(jnp.int32, (NSEG, 1), 0)
    valid = jnp.logical_and(segc >= 1, segc < n)
    score = jnp.where(valid, score, jnp.float32(-10000.0))
    m = jnp.max(score, axis=0, keepdims=True)
    e = jnp.exp(score - m)
    w = e / jnp.sum(e, axis=0, keepdims=True)
    w = w * valid.astype(jnp.float32)
    out_ref[0] = jax.lax.dot_general(
        w, f, (((0,), (0,)), ((), ())),
        preferred_element_type=jnp.float32)


@functools.partial(jax.jit, static_argnames=("interpret",))
def _run(last_hidden_state, labels3, W_w, W_b2, V_w, interpret=False):
    B, S, D = last_hidden_state.shape
    H = W_w.shape[0]

    ctx = pl.pallas_call(
        _fused_kernel,
        grid=(B,),
        in_specs=[
            pl.BlockSpec((1, 1, S), lambda i: (i, 0, 0)),
            pl.BlockSpec((1, S, D), lambda i: (i, 0, 0)),
            pl.BlockSpec((H, D), lambda i: (0, 0)),
            pl.BlockSpec((1, H), lambda i: (0, 0)),
            pl.BlockSpec((1, H), lambda i: (0, 0)),
        ],
        out_specs=pl.BlockSpec((1, 1, D), lambda i: (i, 0, 0)),
        out_shape=jax.ShapeDtypeStruct((B, 1, D), jnp.float32),
        compiler_params=pltpu.CompilerParams(
            dimension_semantics=("arbitrary",)),
        interpret=interpret,
    )(labels3, last_hidden_state, W_w, W_b2, V_w)
    return ctx.reshape(B, D)


def kernel(last_hidden_state, labeled_input_ids, W_w, W_b, V_w, V_b):
    B, S, D = last_hidden_state.shape
    H = W_w.shape[0]
    labels3 = labeled_input_ids.astype(jnp.int32).reshape(B, 1, S)
    return _run(last_hidden_state, labels3, W_w, W_b.reshape(1, H), V_w)
